# baseline (speedup 1.0000x reference)
import functools

import jax
import jax.numpy as jnp
from jax import lax
from jax.experimental import pallas as pl
from jax.experimental.pallas import tpu as pltpu
from jax.experimental.pallas import tpu_sc as plsc

N = 10000
E = 320000
D = 128
DOUT = 64
NT = 16
CHUNK = 128
NCHUNK = 160
SUPER = 16
NSUP = NCHUNK // SUPER
ZC = 128
EPT = NCHUNK * CHUNK
E_PAD = NT * EPT
RPT = 640
NACC = NT * RPT
RB = 2000
NBLK = N // RB

f32 = jnp.float32
i32 = jnp.int32

_mesh = plsc.VectorSubcoreMesh(core_axis_name="c", subcore_axis_name="s")


@functools.partial(
    pl.kernel,
    out_type=(jax.ShapeDtypeStruct((NACC,), f32),
              jax.ShapeDtypeStruct((NACC,), f32)),
    mesh=_mesh,
    scratch_types=[
        pltpu.VMEM_SHARED((NACC,), f32),
        pltpu.VMEM((SUPER * CHUNK,), i32),
        pltpu.VMEM((SUPER * CHUNK,), i32),
        pltpu.VMEM((SUPER * CHUNK,), f32),
        pltpu.VMEM((ZC,), f32),
        pltpu.SemaphoreType.DMA,
        pltpu.SemaphoreType.DMA,
        pltpu.SemaphoreType.DMA,
        pltpu.SemaphoreType.DMA,
    ],
)
def _deg_kernel(dst_p, dst_l, ones_h, zeros_h, deg_p_out, deg_l_out,
                deg_sp, idx_a, idx_b, ones_v, zer_v, ia, ib, sa, sb):
    t = lax.axis_index("s")
    core = lax.axis_index("c")
    base = t * RPT

    pltpu.sync_copy(zeros_h, zer_v)
    for k in range(RPT // ZC):
        pltpu.sync_copy(zer_v, deg_sp.at[pl.ds(base + k * ZC, ZC)])
    plsc.subcore_barrier()
    pltpu.sync_copy(ones_h, ones_v)

    def run(idx_hbm, out_hbm):
        pltpu.async_copy(idx_hbm.at[t].at[0], idx_a, ia)

        @pl.loop(0, NSUP // 2)
        def _(bp):
            pltpu.make_async_copy(idx_hbm.at[t].at[2 * bp], idx_a, ia).wait()
            pltpu.async_copy(idx_hbm.at[t].at[2 * bp + 1], idx_b, ib)
            pltpu.async_copy(ones_v, deg_sp.at[idx_a], sa, add=True)
            pltpu.make_async_copy(idx_hbm.at[t].at[2 * bp + 1], idx_b,
                                  ib).wait()
            pltpu.make_async_copy(ones_v, deg_sp.at[idx_a], sa).wait()

            @pl.when(bp < NSUP // 2 - 1)
            def _():
                pltpu.async_copy(idx_hbm.at[t].at[2 * bp + 2], idx_a, ia)

            pltpu.async_copy(ones_v, deg_sp.at[idx_b], sb, add=True)
            pltpu.make_async_copy(ones_v, deg_sp.at[idx_b], sb).wait()

        plsc.subcore_barrier()
        pltpu.sync_copy(deg_sp.at[pl.ds(base, RPT)],
                        out_hbm.at[pl.ds(base, RPT)])

    @pl.when(core == 0)
    def _():
        run(dst_p, deg_p_out)

    @pl.when(core == 1)
    def _():
        run(dst_l, deg_l_out)


@functools.partial(
    pl.kernel,
    out_type=(jax.ShapeDtypeStruct((NACC, D), f32),
              jax.ShapeDtypeStruct((NACC,), f32),
              jax.ShapeDtypeStruct((NACC, D), f32),
              jax.ShapeDtypeStruct((NACC,), f32)),
    mesh=_mesh,
    scratch_types=[
        pltpu.VMEM_SHARED((NACC, D), f32),
        pltpu.VMEM_SHARED((NACC,), f32),
        pltpu.VMEM((SUPER * CHUNK,), i32),
        pltpu.VMEM((SUPER, CHUNK), i32),
        pltpu.VMEM((SUPER * CHUNK,), i32),
        pltpu.VMEM((SUPER * CHUNK,), i32),
        pltpu.VMEM((SUPER, CHUNK), i32),
        pltpu.VMEM((SUPER * CHUNK,), i32),
        pltpu.VMEM((CHUNK, D), f32),
        pltpu.VMEM((CHUNK, D), f32),
        pltpu.VMEM((SUPER * CHUNK,), f32),
        pltpu.SemaphoreType.DMA,
        pltpu.SemaphoreType.DMA,
        pltpu.SemaphoreType.DMA,
        pltpu.SemaphoreType.DMA,
        pltpu.SemaphoreType.DMA,
        pltpu.SemaphoreType.DMA,
        pltpu.SemaphoreType.DMA,
        pltpu.SemaphoreType.DMA,
    ],
)
def _agg_kernel(ys_p, dinv_p, src_p, dst2_p, dstf_p,
                ys_l, dinv_l, src_l, dst2_l, dstf_l,
                zeros_h,
                S_p_out, s_p_out, S_l_out, s_l_out,
                S_sp, s_sp, src0_v, d20_v, df0_v, src1_v, d21_v, df1_v,
                rows_a, rows_b, vals_v, g0, g1, s0, s1, vg, vs, i0, i1):
    t = lax.axis_index("s")
    core = lax.axis_index("c")
    base = t * RPT

    pltpu.sync_copy(zeros_h, rows_a)
    for k in range(RPT // CHUNK):
        pltpu.sync_copy(rows_a, S_sp.at[pl.ds(base + k * CHUNK, CHUNK)])
    for k in range(RPT // ZC):
        pltpu.sync_copy(rows_a.at[0], s_sp.at[pl.ds(base + k * ZC, ZC)])
    plsc.subcore_barrier()

    def run(ys, dinv, src, dst2, dstf, S_out, s_out):
        set0 = (src0_v, d20_v, df0_v, i0)
        set1 = (src1_v, d21_v, df1_v, i1)
        npair = NSUP // 2

        def stage(sc, st):
            sv, d2, df, sem = st
            pltpu.async_copy(src.at[t].at[sc], sv, sem)
            pltpu.async_copy(dst2.at[t].at[sc], d2, sem)
            pltpu.async_copy(dstf.at[t].at[sc], df, sem)

        def stage_wait(sc, st):
            sv, d2, df, sem = st
            pltpu.make_async_copy(src.at[t].at[sc], sv, sem).wait()
            pltpu.make_async_copy(dst2.at[t].at[sc], d2, sem).wait()
            pltpu.make_async_copy(dstf.at[t].at[sc], df, sem).wait()

        def g_issue(sv, c, buf, sem):
            pltpu.async_copy(ys.at[sv.at[pl.ds(c * CHUNK, CHUNK)]], buf, sem)

        def g_wait(sv, c, buf, sem):
            pltpu.make_async_copy(ys.at[sv.at[pl.ds(c * CHUNK, CHUNK)]], buf,
                                  sem).wait()

        def s_issue(d2, r, buf, sem):
            pltpu.async_copy(buf, S_sp.at[d2.at[r]], sem, add=True)

        def s_wait(d2, r, buf, sem):
            pltpu.make_async_copy(buf, S_sp.at[d2.at[r]], sem).wait()

        def block(st, cross):
            sv, d2, df, _ = st
            pltpu.async_copy(dinv.at[df], vals_v, vg)
            for j in range(SUPER // 2):
                lastj = j == SUPER // 2 - 1
                g_wait(sv, 2 * j, rows_a, g0)
                s_issue(d2, 2 * j, rows_a, s0)
                g_wait(sv, 2 * j + 1, rows_b, g1)
                s_wait(d2, 2 * j, rows_a, s0)
                if not lastj:
                    g_issue(sv, 2 * j + 2, rows_a, g0)
                else:
                    cross(0, rows_a, g0)
                s_issue(d2, 2 * j + 1, rows_b, s1)
                s_wait(d2, 2 * j + 1, rows_b, s1)
                if not lastj:
                    g_issue(sv, 2 * j + 3, rows_b, g1)
                else:
                    cross(1, rows_b, g1)
            pltpu.make_async_copy(dinv.at[df], vals_v, vg).wait()
            pltpu.async_copy(vals_v, s_sp.at[sv], vs, add=True)
            pltpu.make_async_copy(vals_v, s_sp.at[sv], vs).wait()

        stage(0, set0)
        stage(1, set1)
        stage_wait(0, set0)
        g_issue(src0_v, 0, rows_a, g0)
        g_issue(src0_v, 1, rows_b, g1)

        @pl.loop(0, npair)
        def _(bp):
            stage_wait(2 * bp + 1, set1)

            def cross_a(k, buf, sem):
                g_issue(src1_v, k, buf, sem)

            block(set0, cross_a)

            @pl.when(bp < npair - 1)
            def _():
                stage(2 * bp + 2, set0)

            def cross_b(k, buf, sem):
                @pl.when(bp < npair - 1)
                def _():
                    if k == 0:
                        stage_wait(2 * bp + 2, set0)
                    g_issue(src0_v, k, buf, sem)

            block(set1, cross_b)

            @pl.when(bp < npair - 1)
            def _():
                stage(2 * bp + 3, set1)

        plsc.subcore_barrier()
        pltpu.sync_copy(S_sp.at[pl.ds(base, RPT)], S_out.at[pl.ds(base, RPT)])
        pltpu.sync_copy(s_sp.at[pl.ds(base, RPT)], s_out.at[pl.ds(base, RPT)])

    @pl.when(core == 0)
    def _():
        run(ys_p, dinv_p, src_p, dst2_p, dstf_p, S_p_out, s_p_out)

    @pl.when(core == 1)
    def _():
        run(ys_l, dinv_l, src_l, dst2_l, dstf_l, S_l_out, s_l_out)


def _tcmm_body(x_p, w1_p, x_l, w1_l, xw_p, xw_l):
    xw_p[...] = jnp.dot(x_p[...], w1_p[...], preferred_element_type=f32)
    xw_l[...] = jnp.dot(x_l[...], w1_l[...], preferred_element_type=f32)


_tcmm = pl.pallas_call(
    _tcmm_body,
    grid=(NBLK,),
    in_specs=[
        pl.BlockSpec((RB, D), lambda i: (i, 0)),
        pl.BlockSpec((D, D), lambda i: (0, 0)),
        pl.BlockSpec((RB, D), lambda i: (i, 0)),
        pl.BlockSpec((D, D), lambda i: (0, 0)),
    ],
    out_specs=[
        pl.BlockSpec((RB, D), lambda i: (i, 0)),
        pl.BlockSpec((RB, D), lambda i: (i, 0)),
    ],
    out_shape=[
        jax.ShapeDtypeStruct((N, D), f32),
        jax.ShapeDtypeStruct((N, D), f32),
    ],
)


def _tc1_body(xw_p, deg_p, xw_l, deg_l, ys_p, dv_p, ys_l, dv_l):
    def one(xw_ref, deg_ref, ys_ref, dv_ref):
        dinv = lax.rsqrt(deg_ref[...] + 1.0)
        ys_ref[...] = xw_ref[...] * dinv
        dv_ref[...] = dinv

    one(xw_p, deg_p, ys_p, dv_p)
    one(xw_l, deg_l, ys_l, dv_l)


_tc1 = pl.pallas_call(
    _tc1_body,
    grid=(NBLK,),
    in_specs=[
        pl.BlockSpec((RB, D), lambda i: (i, 0)),
        pl.BlockSpec((RB, 1), lambda i: (i, 0)),
        pl.BlockSpec((RB, D), lambda i: (i, 0)),
        pl.BlockSpec((RB, 1), lambda i: (i, 0)),
    ],
    out_specs=[
        pl.BlockSpec((RB, D), lambda i: (i, 0)),
        pl.BlockSpec((RB, 1), lambda i: (i, 0)),
        pl.BlockSpec((RB, D), lambda i: (i, 0)),
        pl.BlockSpec((RB, 1), lambda i: (i, 0)),
    ],
    out_shape=[
        jax.ShapeDtypeStruct((NACC, D), f32),
        jax.ShapeDtypeStruct((NACC, 1), f32),
        jax.ShapeDtypeStruct((NACC, D), f32),
        jax.ShapeDtypeStruct((NACC, 1), f32),
    ],
)


def _tc2_body(S_p, ys_p, dv_p, s_p, b1_p, w2_p, b2_p,
              S_l, ys_l, dv_l, s_l, b1_l, w2_l, b2_l,
              z_p, z_l, acc_p, acc_l):
    i = pl.program_id(0)

    @pl.when(i == 0)
    def _():
        acc_p[...] = jnp.zeros_like(acc_p)
        acc_l[...] = jnp.zeros_like(acc_l)

    def one(S_ref, ys_ref, dv_ref, s_ref, b1_ref, acc):
        dinv = dv_ref[...]
        h = jnp.maximum(dinv * (S_ref[...] + ys_ref[...]) + b1_ref[...], 0.0)
        c = dinv * (dinv + s_ref[...])
        acc[...] += jnp.sum(c * h, axis=0, keepdims=True)

    one(S_p, ys_p, dv_p, s_p, b1_p, acc_p)
    one(S_l, ys_l, dv_l, s_l, b1_l, acc_l)

    @pl.when(i == NBLK - 1)
    def _():
        z_p[...] = (jnp.dot(acc_p[...], w2_p[...], preferred_element_type=f32)
                    * (1.0 / N) + b2_p[...])
        z_l[...] = (jnp.dot(acc_l[...], w2_l[...], preferred_element_type=f32)
                    * (1.0 / N) + b2_l[...])


_tc2 = pl.pallas_call(
    _tc2_body,
    grid=(NBLK,),
    in_specs=[
        pl.BlockSpec((RB, D), lambda i: (i, 0)),
        pl.BlockSpec((RB, D), lambda i: (i, 0)),
        pl.BlockSpec((RB, 1), lambda i: (i, 0)),
        pl.BlockSpec((RB, 1), lambda i: (i, 0)),
        pl.BlockSpec((1, D), lambda i: (0, 0)),
        pl.BlockSpec((D, DOUT), lambda i: (0, 0)),
        pl.BlockSpec((1, DOUT), lambda i: (0, 0)),
    ] * 2,
    out_specs=[
        pl.BlockSpec((1, DOUT), lambda i: (0, 0)),
        pl.BlockSpec((1, DOUT), lambda i: (0, 0)),
    ],
    out_shape=[
        jax.ShapeDtypeStruct((1, DOUT), f32),
        jax.ShapeDtypeStruct((1, DOUT), f32),
    ],
    scratch_shapes=[
        pltpu.VMEM((1, D), f32),
        pltpu.VMEM((1, D), f32),
    ],
)


def kernel(x_p, edge_index_p, x_l, edge_index_l,
           W1_p, b1_p, W2_p, b2_p, W1_l, b1_l, W2_l, b2_l):
    pad = E_PAD - E
    ar = jnp.arange(pad, dtype=i32)
    pads = N + (ar % CHUNK)

    flat = (NT, NSUP, SUPER * CHUNK)
    shp4 = (NT, NSUP, SUPER, CHUNK)

    def prep(ei):
        src = jnp.concatenate([ei[0], pads]).reshape(flat)
        dst = jnp.concatenate([ei[1], pads]).reshape(flat)
        return src, dst

    src_p, dst_p = prep(edge_index_p)
    src_l, dst_l = prep(edge_index_l)

    ones1 = jnp.ones((SUPER * CHUNK,), f32)
    zer1 = jnp.zeros((ZC,), f32)
    zer2 = jnp.zeros((CHUNK, D), f32)

    deg_p, deg_l = _deg_kernel(dst_p, dst_l, ones1, zer1)
    xw_p, xw_l = _tcmm(x_p, W1_p, x_l, W1_l)

    ys_p, dv_p, ys_l, dv_l = _tc1(
        xw_p, deg_p.reshape(NACC, 1),
        xw_l, deg_l.reshape(NACC, 1))

    S_p, s_p, S_l, s_l = _agg_kernel(
        ys_p, dv_p.reshape(NACC), src_p, dst_p.reshape(shp4), dst_p,
        ys_l, dv_l.reshape(NACC), src_l, dst_l.reshape(shp4), dst_l,
        zer2)

    z_p, z_l = _tc2(
        S_p, ys_p, dv_p, s_p.reshape(NACC, 1),
        b1_p.reshape(1, D), W2_p, b2_p.reshape(1, DOUT),
        S_l, ys_l, dv_l, s_l.reshape(NACC, 1),
        b1_l.reshape(1, D), W2_l, b2_l.reshape(1, DOUT))

    return z_p.reshape(DOUT), z_l.reshape(DOUT)


# speedup vs baseline: 1.0192x; 1.0192x over previous
import functools

import jax
import jax.numpy as jnp
from jax import lax
from jax.experimental import pallas as pl
from jax.experimental.pallas import tpu as pltpu
from jax.experimental.pallas import tpu_sc as plsc

N = 10000
E = 320000
D = 128
DOUT = 64
NT = 16
CHUNK = 128
NCHUNK = 160
SUPER = 16
NSUP = NCHUNK // SUPER
ZC = 128
EPT = NCHUNK * CHUNK
E_PAD = NT * EPT
RPT = 640
NACC = NT * RPT
RB = 2000
NBLK = N // RB

f32 = jnp.float32
i32 = jnp.int32

_mesh = plsc.VectorSubcoreMesh(core_axis_name="c", subcore_axis_name="s")


@functools.partial(
    pl.kernel,
    out_type=(jax.ShapeDtypeStruct((NACC,), f32),
              jax.ShapeDtypeStruct((NACC,), f32)),
    mesh=_mesh,
    scratch_types=[
        pltpu.VMEM_SHARED((NACC,), f32),
        pltpu.VMEM((SUPER * CHUNK,), i32),
        pltpu.VMEM((SUPER * CHUNK,), i32),
        pltpu.VMEM((SUPER * CHUNK,), f32),
        pltpu.VMEM((ZC,), f32),
        pltpu.SemaphoreType.DMA,
        pltpu.SemaphoreType.DMA,
        pltpu.SemaphoreType.DMA,
        pltpu.SemaphoreType.DMA,
    ],
)
def _deg_kernel(dst_p, dst_l, ones_h, zeros_h, deg_p_out, deg_l_out,
                deg_sp, idx_a, idx_b, ones_v, zer_v, ia, ib, sa, sb):
    t = lax.axis_index("s")
    core = lax.axis_index("c")
    base = t * RPT

    pltpu.sync_copy(zeros_h, zer_v)
    for k in range(RPT // ZC):
        pltpu.sync_copy(zer_v, deg_sp.at[pl.ds(base + k * ZC, ZC)])
    plsc.subcore_barrier()
    pltpu.sync_copy(ones_h, ones_v)

    def run(idx_hbm, out_hbm):
        pltpu.async_copy(idx_hbm.at[t].at[0], idx_a, ia)

        @pl.loop(0, NSUP // 2)
        def _(bp):
            pltpu.make_async_copy(idx_hbm.at[t].at[2 * bp], idx_a, ia).wait()
            pltpu.async_copy(idx_hbm.at[t].at[2 * bp + 1], idx_b, ib)
            pltpu.async_copy(ones_v, deg_sp.at[idx_a], sa, add=True)
            pltpu.make_async_copy(idx_hbm.at[t].at[2 * bp + 1], idx_b,
                                  ib).wait()
            pltpu.make_async_copy(ones_v, deg_sp.at[idx_a], sa).wait()

            @pl.when(bp < NSUP // 2 - 1)
            def _():
                pltpu.async_copy(idx_hbm.at[t].at[2 * bp + 2], idx_a, ia)

            pltpu.async_copy(ones_v, deg_sp.at[idx_b], sb, add=True)
            pltpu.make_async_copy(ones_v, deg_sp.at[idx_b], sb).wait()

        plsc.subcore_barrier()
        pltpu.sync_copy(deg_sp.at[pl.ds(base, RPT)],
                        out_hbm.at[pl.ds(base, RPT)])

    @pl.when(core == 0)
    def _():
        run(dst_p, deg_p_out)

    @pl.when(core == 1)
    def _():
        run(dst_l, deg_l_out)


@functools.partial(
    pl.kernel,
    out_type=(jax.ShapeDtypeStruct((NACC, D), f32),
              jax.ShapeDtypeStruct((NACC,), f32),
              jax.ShapeDtypeStruct((NACC, D), f32),
              jax.ShapeDtypeStruct((NACC,), f32)),
    mesh=_mesh,
    scratch_types=[
        pltpu.VMEM_SHARED((NACC, D), f32),
        pltpu.VMEM_SHARED((NACC,), f32),
        pltpu.VMEM((SUPER * CHUNK,), i32),
        pltpu.VMEM((SUPER, CHUNK), i32),
        pltpu.VMEM((SUPER * CHUNK,), i32),
        pltpu.VMEM((SUPER * CHUNK,), i32),
        pltpu.VMEM((SUPER, CHUNK), i32),
        pltpu.VMEM((SUPER * CHUNK,), i32),
        pltpu.VMEM((CHUNK, D), f32),
        pltpu.VMEM((CHUNK, D), f32),
        pltpu.VMEM((SUPER * CHUNK,), f32),
        pltpu.SemaphoreType.DMA,
        pltpu.SemaphoreType.DMA,
        pltpu.SemaphoreType.DMA,
        pltpu.SemaphoreType.DMA,
        pltpu.SemaphoreType.DMA,
        pltpu.SemaphoreType.DMA,
        pltpu.SemaphoreType.DMA,
        pltpu.SemaphoreType.DMA,
    ],
)
def _agg_kernel(ys_p, dinv_p, src_p, dst2_p, dstf_p,
                ys_l, dinv_l, src_l, dst2_l, dstf_l,
                zeros_h,
                S_p_out, s_p_out, S_l_out, s_l_out,
                S_sp, s_sp, src0_v, d20_v, df0_v, src1_v, d21_v, df1_v,
                rows_a, rows_b, vals_v, g0, g1, s0, s1, vg, vs, i0, i1):
    t = lax.axis_index("s")
    core = lax.axis_index("c")
    base = t * RPT

    pltpu.sync_copy(zeros_h, rows_a)
    for k in range(RPT // CHUNK):
        pltpu.sync_copy(rows_a, S_sp.at[pl.ds(base + k * CHUNK, CHUNK)])
    for k in range(RPT // ZC):
        pltpu.sync_copy(rows_a.at[0], s_sp.at[pl.ds(base + k * ZC, ZC)])
    plsc.subcore_barrier()

    def run(ys, dinv, src, dst2, dstf, S_out, s_out):
        set0 = (src0_v, d20_v, df0_v, i0)
        set1 = (src1_v, d21_v, df1_v, i1)
        npair = NSUP // 2

        def stage(sc, st):
            sv, d2, df, sem = st
            pltpu.async_copy(src.at[t].at[sc], sv, sem)
            pltpu.async_copy(dst2.at[t].at[sc], d2, sem)
            pltpu.async_copy(dstf.at[t].at[sc], df, sem)

        def stage_wait(sc, st):
            sv, d2, df, sem = st
            pltpu.make_async_copy(src.at[t].at[sc], sv, sem).wait()
            pltpu.make_async_copy(dst2.at[t].at[sc], d2, sem).wait()
            pltpu.make_async_copy(dstf.at[t].at[sc], df, sem).wait()

        def g_issue(sv, c, buf, sem):
            pltpu.async_copy(ys.at[sv.at[pl.ds(c * CHUNK, CHUNK)]], buf, sem)

        def g_wait(sv, c, buf, sem):
            pltpu.make_async_copy(ys.at[sv.at[pl.ds(c * CHUNK, CHUNK)]], buf,
                                  sem).wait()

        def s_issue(d2, r, buf, sem):
            pltpu.async_copy(buf, S_sp.at[d2.at[r]], sem, add=True)

        def s_wait(d2, r, buf, sem):
            pltpu.make_async_copy(buf, S_sp.at[d2.at[r]], sem).wait()

        def block(st, cross):
            sv, d2, df, _ = st
            pltpu.async_copy(dinv.at[df], vals_v, vg)
            for j in range(SUPER // 2):
                lastj = j == SUPER // 2 - 1
                g_wait(sv, 2 * j, rows_a, g0)
                s_issue(d2, 2 * j, rows_a, s0)
                g_wait(sv, 2 * j + 1, rows_b, g1)
                s_wait(d2, 2 * j, rows_a, s0)
                if not lastj:
                    g_issue(sv, 2 * j + 2, rows_a, g0)
                else:
                    cross(0, rows_a, g0)
                s_issue(d2, 2 * j + 1, rows_b, s1)
                s_wait(d2, 2 * j + 1, rows_b, s1)
                if not lastj:
                    g_issue(sv, 2 * j + 3, rows_b, g1)
                else:
                    cross(1, rows_b, g1)
            pltpu.make_async_copy(dinv.at[df], vals_v, vg).wait()
            pltpu.async_copy(vals_v, s_sp.at[sv], vs, add=True)
            pltpu.make_async_copy(vals_v, s_sp.at[sv], vs).wait()

        stage(0, set0)
        stage(1, set1)
        stage_wait(0, set0)
        g_issue(src0_v, 0, rows_a, g0)
        g_issue(src0_v, 1, rows_b, g1)

        @pl.loop(0, npair)
        def _(bp):
            stage_wait(2 * bp + 1, set1)

            def cross_a(k, buf, sem):
                g_issue(src1_v, k, buf, sem)

            block(set0, cross_a)

            @pl.when(bp < npair - 1)
            def _():
                stage(2 * bp + 2, set0)

            def cross_b(k, buf, sem):
                @pl.when(bp < npair - 1)
                def _():
                    if k == 0:
                        stage_wait(2 * bp + 2, set0)
                    g_issue(src0_v, k, buf, sem)

            block(set1, cross_b)

            @pl.when(bp < npair - 1)
            def _():
                stage(2 * bp + 3, set1)

        plsc.subcore_barrier()
        pltpu.sync_copy(S_sp.at[pl.ds(base, RPT)], S_out.at[pl.ds(base, RPT)])
        pltpu.sync_copy(s_sp.at[pl.ds(base, RPT)], s_out.at[pl.ds(base, RPT)])

    @pl.when(core == 0)
    def _():
        run(ys_p, dinv_p, src_p, dst2_p, dstf_p, S_p_out, s_p_out)

    @pl.when(core == 1)
    def _():
        run(ys_l, dinv_l, src_l, dst2_l, dstf_l, S_l_out, s_l_out)


def _tc1_body(x_p, w1_p, deg_p, x_l, w1_l, deg_l, ys_p, dv_p, ys_l, dv_l):
    def one(x_ref, w_ref, deg_ref, ys_ref, dv_ref):
        dinv = lax.rsqrt(deg_ref[...] + 1.0)
        ys_ref[...] = jnp.dot(x_ref[...], w_ref[...],
                              preferred_element_type=f32) * dinv
        dv_ref[...] = dinv

    one(x_p, w1_p, deg_p, ys_p, dv_p)
    one(x_l, w1_l, deg_l, ys_l, dv_l)


_tc1 = pl.pallas_call(
    _tc1_body,
    grid=(NBLK,),
    in_specs=[
        pl.BlockSpec((RB, D), lambda i: (i, 0)),
        pl.BlockSpec((D, D), lambda i: (0, 0)),
        pl.BlockSpec((RB, 1), lambda i: (i, 0)),
        pl.BlockSpec((RB, D), lambda i: (i, 0)),
        pl.BlockSpec((D, D), lambda i: (0, 0)),
        pl.BlockSpec((RB, 1), lambda i: (i, 0)),
    ],
    out_specs=[
        pl.BlockSpec((RB, D), lambda i: (i, 0)),
        pl.BlockSpec((RB, 1), lambda i: (i, 0)),
        pl.BlockSpec((RB, D), lambda i: (i, 0)),
        pl.BlockSpec((RB, 1), lambda i: (i, 0)),
    ],
    out_shape=[
        jax.ShapeDtypeStruct((NACC, D), f32),
        jax.ShapeDtypeStruct((NACC, 1), f32),
        jax.ShapeDtypeStruct((NACC, D), f32),
        jax.ShapeDtypeStruct((NACC, 1), f32),
    ],
)


def _tc2_body(S_p, ys_p, dv_p, s_p, b1_p, w2_p, b2_p,
              S_l, ys_l, dv_l, s_l, b1_l, w2_l, b2_l,
              z_p, z_l, acc_p, acc_l):
    i = pl.program_id(0)

    @pl.when(i == 0)
    def _():
        acc_p[...] = jnp.zeros_like(acc_p)
        acc_l[...] = jnp.zeros_like(acc_l)

    def one(S_ref, ys_ref, dv_ref, s_ref, b1_ref, acc):
        dinv = dv_ref[...]
        h = jnp.maximum(dinv * (S_ref[...] + ys_ref[...]) + b1_ref[...], 0.0)
        c = dinv * (dinv + s_ref[...])
        acc[...] += jnp.sum(c * h, axis=0, keepdims=True)

    one(S_p, ys_p, dv_p, s_p, b1_p, acc_p)
    one(S_l, ys_l, dv_l, s_l, b1_l, acc_l)

    @pl.when(i == NBLK - 1)
    def _():
        z_p[...] = (jnp.dot(acc_p[...], w2_p[...], preferred_element_type=f32)
                    * (1.0 / N) + b2_p[...])
        z_l[...] = (jnp.dot(acc_l[...], w2_l[...], preferred_element_type=f32)
                    * (1.0 / N) + b2_l[...])


_tc2 = pl.pallas_call(
    _tc2_body,
    grid=(NBLK,),
    in_specs=[
        pl.BlockSpec((RB, D), lambda i: (i, 0)),
        pl.BlockSpec((RB, D), lambda i: (i, 0)),
        pl.BlockSpec((RB, 1), lambda i: (i, 0)),
        pl.BlockSpec((RB, 1), lambda i: (i, 0)),
        pl.BlockSpec((1, D), lambda i: (0, 0)),
        pl.BlockSpec((D, DOUT), lambda i: (0, 0)),
        pl.BlockSpec((1, DOUT), lambda i: (0, 0)),
    ] * 2,
    out_specs=[
        pl.BlockSpec((1, DOUT), lambda i: (0, 0)),
        pl.BlockSpec((1, DOUT), lambda i: (0, 0)),
    ],
    out_shape=[
        jax.ShapeDtypeStruct((1, DOUT), f32),
        jax.ShapeDtypeStruct((1, DOUT), f32),
    ],
    scratch_shapes=[
        pltpu.VMEM((1, D), f32),
        pltpu.VMEM((1, D), f32),
    ],
)


def kernel(x_p, edge_index_p, x_l, edge_index_l,
           W1_p, b1_p, W2_p, b2_p, W1_l, b1_l, W2_l, b2_l):
    pad = E_PAD - E
    ar = jnp.arange(pad, dtype=i32)
    pads = N + (ar % CHUNK)

    flat = (NT, NSUP, SUPER * CHUNK)
    shp4 = (NT, NSUP, SUPER, CHUNK)

    def prep(ei):
        src = jnp.concatenate([ei[0], pads]).reshape(flat)
        dst = jnp.concatenate([ei[1], pads]).reshape(flat)
        return src, dst

    src_p, dst_p = prep(edge_index_p)
    src_l, dst_l = prep(edge_index_l)

    ones1 = jnp.ones((SUPER * CHUNK,), f32)
    zer1 = jnp.zeros((ZC,), f32)
    zer2 = jnp.zeros((CHUNK, D), f32)

    deg_p, deg_l = _deg_kernel(dst_p, dst_l, ones1, zer1)

    ys_p, dv_p, ys_l, dv_l = _tc1(
        x_p, W1_p, deg_p.reshape(NACC, 1),
        x_l, W1_l, deg_l.reshape(NACC, 1))

    S_p, s_p, S_l, s_l = _agg_kernel(
        ys_p, dv_p.reshape(NACC), src_p, dst_p.reshape(shp4), dst_p,
        ys_l, dv_l.reshape(NACC), src_l, dst_l.reshape(shp4), dst_l,
        zer2)

    z_p, z_l = _tc2(
        S_p, ys_p, dv_p, s_p.reshape(NACC, 1),
        b1_p.reshape(1, D), W2_p, b2_p.reshape(1, DOUT),
        S_l, ys_l, dv_l, s_l.reshape(NACC, 1),
        b1_l.reshape(1, D), W2_l, b2_l.reshape(1, DOUT))

    return z_p.reshape(DOUT), z_l.reshape(DOUT)
